# baseline (device time: 21488 ns/iter reference)
import jax
import jax.numpy as jnp
from jax import lax
from jax.experimental import pallas as pl
from jax.experimental.pallas import tpu as pltpu

N_DEV = 4
K_CHUNKS = 8


def kernel(x, pi):
    _, m, n = x.shape
    rows = m // K_CHUNKS

    def body(
        pi_ref, x_ref, out_ref,
        q_send, q_recv, sc_send, sc_recv,
        dsend_sems, drecv_sems, ssend_sems, srecv_sems,
    ):
        my_pos = lax.axis_index("i")
        dst = pi_ref[my_pos]
        src = jnp.int32(0)
        for j in range(N_DEV):
            src = jnp.where(pi_ref[j] == my_pos, jnp.int32(j), src)

        barrier_sem = pltpu.get_barrier_semaphore()
        pl.semaphore_signal(
            barrier_sem, inc=1,
            device_id=(src,), device_id_type=pl.DeviceIdType.MESH,
        )
        pl.semaphore_wait(barrier_sem, 1)

        rdmas = []
        for k in range(K_CHUNKS):
            sl = pl.ds(k * rows, rows)
            chunk = x_ref[0, sl, :]
            mx = jnp.maximum(jnp.max(jnp.abs(chunk)), 1e-30)
            scale = mx / 127.0
            q_send[sl, :] = jnp.round(chunk * (127.0 / mx)).astype(jnp.int8)
            sc_send[k : k + 1, :] = jnp.broadcast_to(
                scale[jnp.newaxis, jnp.newaxis], (1, 128)
            )
            d_rdma = pltpu.make_async_remote_copy(
                src_ref=q_send.at[sl, :],
                dst_ref=q_recv.at[sl, :],
                send_sem=dsend_sems.at[k],
                recv_sem=drecv_sems.at[k],
                device_id=(dst,),
                device_id_type=pl.DeviceIdType.MESH,
            )
            d_rdma.start()
            s_rdma = pltpu.make_async_remote_copy(
                src_ref=sc_send.at[k : k + 1, :],
                dst_ref=sc_recv.at[k : k + 1, :],
                send_sem=ssend_sems.at[k],
                recv_sem=srecv_sems.at[k],
                device_id=(dst,),
                device_id_type=pl.DeviceIdType.MESH,
            )
            s_rdma.start()
            rdmas.append((d_rdma, s_rdma))

        for k in range(K_CHUNKS):
            sl = pl.ds(k * rows, rows)
            d_rdma, s_rdma = rdmas[k]
            d_rdma.wait_recv()
            s_rdma.wait_recv()
            out_ref[0, sl, :] = (
                q_recv[sl, :].astype(jnp.float32) * sc_recv[k : k + 1, 0:1]
            )

        for d_rdma, s_rdma in rdmas:
            d_rdma.wait_send()
            s_rdma.wait_send()

    return pl.pallas_call(
        body,
        out_shape=jax.ShapeDtypeStruct((1, m, n), jnp.float32),
        in_specs=[
            pl.BlockSpec(memory_space=pltpu.SMEM),
            pl.BlockSpec(memory_space=pltpu.VMEM),
        ],
        out_specs=pl.BlockSpec(memory_space=pltpu.VMEM),
        scratch_shapes=[
            pltpu.VMEM((m, n), jnp.int8),
            pltpu.VMEM((m, n), jnp.int8),
            pltpu.VMEM((K_CHUNKS, 128), jnp.float32),
            pltpu.VMEM((K_CHUNKS, 128), jnp.float32),
            pltpu.SemaphoreType.DMA((K_CHUNKS,)),
            pltpu.SemaphoreType.DMA((K_CHUNKS,)),
            pltpu.SemaphoreType.DMA((K_CHUNKS,)),
            pltpu.SemaphoreType.DMA((K_CHUNKS,)),
        ],
        compiler_params=pltpu.CompilerParams(collective_id=0),
    )(pi, x)
